# baseline (speedup 1.0000x reference)
import functools
import math

import jax
import jax.numpy as jnp
from jax import lax
from jax.experimental import pallas as pl
from jax.experimental.pallas import tpu as pltpu
from jax.experimental.pallas import tpu_sc as plsc

D_MODEL = 64
SCALE = math.sqrt(D_MODEL)
LANES = 16

_info = plsc.get_sparse_core_info()
NC, NS = _info.num_cores, _info.num_subcores
NW = NC * NS


def _emb_body(n_rows, chunk, table_hbm, idx_hbm, out_hbm, idx_v, rows_v, sem):
    wid = lax.axis_index("s") * NC + lax.axis_index("c")
    base = wid * n_rows
    n_chunks = n_rows // chunk

    pltpu.sync_copy(idx_hbm.at[pl.ds(base, n_rows)], idx_v)

    def do_chunk(i, carry):
        off = i * chunk
        pltpu.async_copy(table_hbm.at[idx_v.at[pl.ds(off, chunk)]],
                         rows_v, sem).wait()

        def scale_row(r, c):
            for l in range(D_MODEL // LANES):
                s = pl.ds(l * LANES, LANES)
                rows_v[r, s] = rows_v[r, s] * SCALE
            return c

        lax.fori_loop(0, chunk, scale_row, 0, unroll=4)
        pltpu.sync_copy(rows_v, out_hbm.at[pl.ds(base + off, chunk)])
        return carry

    lax.fori_loop(0, n_chunks, do_chunk, 0)


def kernel(x, lut):
    b, s = x.shape
    n = b * s
    idx = x.reshape(n).astype(jnp.int32)
    n_rows = n // NW
    chunk = 1024

    body = functools.partial(_emb_body, n_rows, chunk)
    out = pl.kernel(
        body,
        out_type=jax.ShapeDtypeStruct((n, D_MODEL), jnp.float32),
        mesh=plsc.VectorSubcoreMesh(core_axis_name="c", subcore_axis_name="s"),
        compiler_params=pltpu.CompilerParams(use_tc_tiling_on_sc=False),
        scratch_types=[
            pltpu.VMEM((n_rows,), jnp.int32),
            pltpu.VMEM((chunk, D_MODEL), jnp.float32),
            pltpu.SemaphoreType.DMA,
        ],
    )(lut, idx)
    return out.reshape(b, s, D_MODEL)
